# baseline (speedup 1.0000x reference)
import functools

import jax
import jax.numpy as jnp
import numpy as np
from jax import lax
from jax.experimental import pallas as pl
from jax.experimental.pallas import tpu as pltpu
from jax.experimental.pallas import tpu_sc as plsc

N = 10000
E = 160000
D = 256
H = 64
HW = H // 2
NP = D // H
NC = 2
NS = 16
B = 128
TPW = 10240
EP = NS * TPW
NB = TPW // B
IGRP = 16
ACC_R = 10240
RPT = ACC_R // NS


def _sc_aggregate(gidx, sidx, fchunks, zrows, ones16):
  mesh = plsc.VectorSubcoreMesh(core_axis_name="c", subcore_axis_name="s")

  @functools.partial(
      pl.kernel,
      out_type=[
          jax.ShapeDtypeStruct((NC, NP, ACC_R, H), jnp.float32),
          jax.ShapeDtypeStruct((NC, ACC_R, 16), jnp.float32),
      ],
      mesh=mesh,
      compiler_params=pltpu.CompilerParams(use_tc_tiling_on_sc=False,
                                           needs_layout_passes=False),
      scratch_types=[
          pltpu.VMEM_SHARED((ACC_R, H), jnp.float32),
          pltpu.VMEM_SHARED((ACC_R, 16), jnp.float32),
          pltpu.VMEM((IGRP, B), jnp.int32),
          pltpu.VMEM((IGRP, B), jnp.int32),
          pltpu.VMEM((B, HW), jnp.int32),
          pltpu.VMEM((B, HW), jnp.int32),
          pltpu.VMEM((B, H), jnp.float32),
          pltpu.VMEM((B, H), jnp.float32),
          pltpu.VMEM((B, 16), jnp.float32),
          pltpu.SemaphoreType.DMA,
          pltpu.SemaphoreType.DMA,
          pltpu.SemaphoreType.DMA,
      ],
  )
  def body(gidx_h, sidx_h, f0_h, f1_h, f2_h, f3_h, zrows_h, ones_h,
           aggs_o, cnts_o, acc_sh, cnt_sh, idxg_all, idxs_all,
           bf0, bf1, fr0, fr1, ones_v, gsem, ssem, csem):
    c = lax.axis_index("c")
    s = lax.axis_index("s")
    rbase = s * RPT

    pltpu.sync_copy(ones_h, ones_v)

    def upconvert(bfb, frb):
      @pl.loop(0, B, unroll=4)
      def conv(r):
        for k in range(HW // 16):
          w = bfb[r, pl.ds(16 * k, 16)]
          lo = plsc.bitcast(w << 16, jnp.float32)
          hi = plsc.bitcast(w & jnp.int32(-65536), jnp.float32)
          frb[r, pl.ds(32 * k, 16)] = lo
          frb[r, pl.ds(32 * k + 16, 16)] = hi

    for h, fsrc in enumerate((f0_h, f1_h, f2_h, f3_h)):
      pltpu.sync_copy(zrows_h, fr0)
      for j in range(RPT // B):
        pltpu.sync_copy(fr0, acc_sh.at[pl.ds(rbase + j * B, B)])
        if h == 0:
          pltpu.sync_copy(fr0.at[pl.ds(0, B), pl.ds(0, 16)],
                          cnt_sh.at[pl.ds(rbase + j * B, B)])
      plsc.subcore_barrier()

      @pl.loop(0, NB // IGRP)
      def igrp_loop(ig):
        bbase = s * NB + ig * IGRP
        pltpu.sync_copy(gidx_h.at[c, pl.ds(bbase, IGRP)], idxg_all)
        pltpu.sync_copy(sidx_h.at[c, pl.ds(bbase, IGRP)], idxs_all)

        @pl.loop(0, IGRP, step=2)
        def grp(j):
          g0 = pltpu.async_copy(fsrc.at[idxg_all.at[j]], bf0, gsem)
          g1 = pltpu.async_copy(fsrc.at[idxg_all.at[j + 1]], bf1, gsem)
          scps = []
          for bfb, frb, jj, gcp in ((bf0, fr0, j, g0), (bf1, fr1, j + 1, g1)):
            gcp.wait()
            upconvert(bfb, frb)
            scps.append(pltpu.async_copy(
                frb, acc_sh.at[idxs_all.at[jj]], ssem, add=True))
            if h == 0:
              scps.append(pltpu.async_copy(
                  ones_v, cnt_sh.at[idxs_all.at[jj]], csem, add=True))
          for cp in scps:
            cp.wait()

      plsc.subcore_barrier()
      pltpu.sync_copy(acc_sh.at[pl.ds(rbase, RPT)],
                      aggs_o.at[c, h, pl.ds(rbase, RPT)])

    pltpu.sync_copy(cnt_sh.at[pl.ds(rbase, RPT)],
                    cnts_o.at[c, pl.ds(rbase, RPT)])

  return body(gidx, sidx, *fchunks, zrows, ones16)


def _tc_combine(aggs, cnts, Wt, bstack):
  RB = 256
  grid = (ACC_R // RB,)

  def body(agg_ref, cnt_ref, wt_ref, b_ref, out_ref):
    cf = cnt_ref[0, :, 0:1]
    cb = cnt_ref[1, :, 0:1]
    acc = cf * b_ref[0:1, :] + cb * b_ref[1:2, :]
    for ci in range(NC):
      for h in range(NP):
        acc += jnp.dot(agg_ref[ci, h],
                       wt_ref[(ci * NP + h) * H:(ci * NP + h + 1) * H],
                       preferred_element_type=jnp.float32)
    deg = cf + cb
    deg = jnp.where(deg == 0.0, 1.0, deg)
    out_ref[...] = jnp.maximum(acc / deg, 0.0)

  return pl.pallas_call(
      body,
      grid=grid,
      in_specs=[
          pl.BlockSpec((NC, NP, RB, H), lambda i: (0, 0, i, 0)),
          pl.BlockSpec((NC, RB, 16), lambda i: (0, i, 0)),
          pl.BlockSpec((2 * D, D), lambda i: (0, 0)),
          pl.BlockSpec((2, D), lambda i: (0, 0)),
      ],
      out_specs=pl.BlockSpec((RB, D), lambda i: (i, 0)),
      out_shape=jax.ShapeDtypeStruct((ACC_R, D), jnp.float32),
  )(aggs, cnts, Wt, bstack)


def _col_perm():
  p = []
  for k in range(H // 32):
    base = 32 * k
    p += [base + 2 * j for j in range(16)]
    p += [base + 2 * j + 1 for j in range(16)]
  return np.array(p)


def kernel(feat, edge_index, Wf, bf, Wb, bb):
  src = edge_index[0]
  dst = edge_index[1]
  npad = EP - E
  pad0 = jnp.zeros((npad,), jnp.int32)
  padN = jnp.full((npad,), N, jnp.int32)
  gidx = jnp.stack([jnp.concatenate([src, pad0]),
                    jnp.concatenate([dst, pad0])]).reshape(NC, EP // B, B)
  sidx = jnp.stack([jnp.concatenate([dst, padN]),
                    jnp.concatenate([src, padN])]).reshape(NC, EP // B, B)
  featb = feat.astype(jnp.bfloat16)
  fchunks = [
      lax.bitcast_convert_type(
          featb[:, i * H:(i + 1) * H].reshape(N, HW, 2), jnp.int32)
      for i in range(NP)
  ]
  zrows = jnp.zeros((B, H), jnp.float32)
  ones16 = jnp.ones((B, 16), jnp.float32)

  aggs, cnts = _sc_aggregate(gidx, sidx, fchunks, zrows, ones16)

  perm = _col_perm()
  row_order = np.concatenate(
      [ci * D + h * H + perm for ci in range(NC) for h in range(NP)])
  Wt = jnp.concatenate([Wf.T, Wb.T], axis=0)[row_order]
  bstack = jnp.stack([bf, bb])
  out = _tc_combine(aggs, cnts, Wt, bstack)
  return out[:N]


# speedup vs baseline: 1.0726x; 1.0726x over previous
import functools

import jax
import jax.numpy as jnp
import numpy as np
from jax import lax
from jax.experimental import pallas as pl
from jax.experimental.pallas import tpu as pltpu
from jax.experimental.pallas import tpu_sc as plsc

N = 10000
E = 160000
D = 256
H = 128
HW = H // 2
NP = D // H
NC = 2
NS = 16
B = 64
TPW = 10240
EP = NS * TPW
NB = TPW // B
IGRP = 16
ACC_R = 10240
RPT = ACC_R // NS


def _sc_aggregate(gidx, sidx, fchunks, zrows, ones16):
  mesh = plsc.VectorSubcoreMesh(core_axis_name="c", subcore_axis_name="s")

  @functools.partial(
      pl.kernel,
      out_type=[
          jax.ShapeDtypeStruct((NC, NP, ACC_R, H), jnp.float32),
          jax.ShapeDtypeStruct((NC, ACC_R, 16), jnp.float32),
      ],
      mesh=mesh,
      compiler_params=pltpu.CompilerParams(use_tc_tiling_on_sc=False,
                                           needs_layout_passes=False),
      scratch_types=[
          pltpu.VMEM_SHARED((ACC_R, H), jnp.float32),
          pltpu.VMEM_SHARED((ACC_R, 16), jnp.float32),
          pltpu.VMEM((IGRP, B), jnp.int32),
          pltpu.VMEM((IGRP, B), jnp.int32),
          pltpu.VMEM((B, HW), jnp.int32),
          pltpu.VMEM((B, HW), jnp.int32),
          pltpu.VMEM((B, H), jnp.float32),
          pltpu.VMEM((B, H), jnp.float32),
          pltpu.VMEM((B, 16), jnp.float32),
          pltpu.SemaphoreType.DMA,
          pltpu.SemaphoreType.DMA,
          pltpu.SemaphoreType.DMA,
      ],
  )
  def body(gidx_h, sidx_h, f0_h, f1_h, zrows_h, ones_h,
           aggs_o, cnts_o, acc_sh, cnt_sh, idxg_all, idxs_all,
           bf0, bf1, fr0, fr1, ones_v, gsem, ssem, csem):
    c = lax.axis_index("c")
    s = lax.axis_index("s")
    rbase = s * RPT

    pltpu.sync_copy(ones_h, ones_v)

    def upconvert(bfb, frb):
      @pl.loop(0, B, unroll=4)
      def conv(r):
        for k in range(HW // 16):
          w = bfb[r, pl.ds(16 * k, 16)]
          lo = plsc.bitcast(w << 16, jnp.float32)
          hi = plsc.bitcast(w & jnp.int32(-65536), jnp.float32)
          frb[r, pl.ds(32 * k, 16)] = lo
          frb[r, pl.ds(32 * k + 16, 16)] = hi

    for h, fsrc in enumerate((f0_h, f1_h)):
      pltpu.sync_copy(zrows_h, fr0)
      for j in range(RPT // B):
        pltpu.sync_copy(fr0, acc_sh.at[pl.ds(rbase + j * B, B)])
        if h == 0:
          pltpu.sync_copy(fr0.at[pl.ds(0, B), pl.ds(0, 16)],
                          cnt_sh.at[pl.ds(rbase + j * B, B)])
      plsc.subcore_barrier()

      @pl.loop(0, NB // IGRP)
      def igrp_loop(ig):
        bbase = s * NB + ig * IGRP
        pltpu.sync_copy(gidx_h.at[c, pl.ds(bbase, IGRP)], idxg_all)
        pltpu.sync_copy(sidx_h.at[c, pl.ds(bbase, IGRP)], idxs_all)

        @pl.loop(0, IGRP, step=2)
        def grp(j):
          g0 = pltpu.async_copy(fsrc.at[idxg_all.at[j]], bf0, gsem)
          g1 = pltpu.async_copy(fsrc.at[idxg_all.at[j + 1]], bf1, gsem)
          scps = []
          for bfb, frb, jj, gcp in ((bf0, fr0, j, g0), (bf1, fr1, j + 1, g1)):
            gcp.wait()
            upconvert(bfb, frb)
            scps.append(pltpu.async_copy(
                frb, acc_sh.at[idxs_all.at[jj]], ssem, add=True))
            if h == 0:
              scps.append(pltpu.async_copy(
                  ones_v, cnt_sh.at[idxs_all.at[jj]], csem, add=True))
          for cp in scps:
            cp.wait()

      plsc.subcore_barrier()
      pltpu.sync_copy(acc_sh.at[pl.ds(rbase, RPT)],
                      aggs_o.at[c, h, pl.ds(rbase, RPT)])

    pltpu.sync_copy(cnt_sh.at[pl.ds(rbase, RPT)],
                    cnts_o.at[c, pl.ds(rbase, RPT)])

  return body(gidx, sidx, *fchunks, zrows, ones16)


def _tc_combine(aggs, cnts, Wt, bstack):
  RB = 256
  grid = (ACC_R // RB,)

  def body(agg_ref, cnt_ref, wt_ref, b_ref, out_ref):
    cf = cnt_ref[0, :, 0:1]
    cb = cnt_ref[1, :, 0:1]
    acc = cf * b_ref[0:1, :] + cb * b_ref[1:2, :]
    for ci in range(NC):
      for h in range(NP):
        acc += jnp.dot(agg_ref[ci, h],
                       wt_ref[(ci * NP + h) * H:(ci * NP + h + 1) * H],
                       preferred_element_type=jnp.float32)
    deg = cf + cb
    deg = jnp.where(deg == 0.0, 1.0, deg)
    out_ref[...] = jnp.maximum(acc / deg, 0.0)

  return pl.pallas_call(
      body,
      grid=grid,
      in_specs=[
          pl.BlockSpec((NC, NP, RB, H), lambda i: (0, 0, i, 0)),
          pl.BlockSpec((NC, RB, 16), lambda i: (0, i, 0)),
          pl.BlockSpec((2 * D, D), lambda i: (0, 0)),
          pl.BlockSpec((2, D), lambda i: (0, 0)),
      ],
      out_specs=pl.BlockSpec((RB, D), lambda i: (i, 0)),
      out_shape=jax.ShapeDtypeStruct((ACC_R, D), jnp.float32),
  )(aggs, cnts, Wt, bstack)


def _col_perm():
  p = []
  for k in range(H // 32):
    base = 32 * k
    p += [base + 2 * j for j in range(16)]
    p += [base + 2 * j + 1 for j in range(16)]
  return np.array(p)


def kernel(feat, edge_index, Wf, bf, Wb, bb):
  src = edge_index[0]
  dst = edge_index[1]
  npad = EP - E
  pad0 = jnp.zeros((npad,), jnp.int32)
  padN = jnp.full((npad,), N, jnp.int32)
  gidx = jnp.stack([jnp.concatenate([src, pad0]),
                    jnp.concatenate([dst, pad0])]).reshape(NC, EP // B, B)
  sidx = jnp.stack([jnp.concatenate([dst, padN]),
                    jnp.concatenate([src, padN])]).reshape(NC, EP // B, B)
  featb = feat.astype(jnp.bfloat16)
  fchunks = [
      lax.bitcast_convert_type(
          featb[:, i * H:(i + 1) * H].reshape(N, HW, 2), jnp.int32)
      for i in range(NP)
  ]
  zrows = jnp.zeros((B, H), jnp.float32)
  ones16 = jnp.ones((B, 16), jnp.float32)

  aggs, cnts = _sc_aggregate(gidx, sidx, fchunks, zrows, ones16)

  perm = _col_perm()
  row_order = np.concatenate(
      [ci * D + h * H + perm for ci in range(NC) for h in range(NP)])
  Wt = jnp.concatenate([Wf.T, Wb.T], axis=0)[row_order]
  bstack = jnp.stack([bf, bb])
  out = _tc_combine(aggs, cnts, Wt, bstack)
  return out[:N]


# speedup vs baseline: 1.1603x; 1.0817x over previous
import functools

import jax
import jax.numpy as jnp
import numpy as np
from jax import lax
from jax.experimental import pallas as pl
from jax.experimental.pallas import tpu as pltpu
from jax.experimental.pallas import tpu_sc as plsc

N = 10000
E = 160000
D = 256
H = 128
HW = H // 2
NP = D // H
NC = 2
NS = 16
B = 128
TPW = 10240
EP = NS * TPW
NB = TPW // B
IGRP = 16
ACC_R = 10240
RPT = ACC_R // NS
CR = ACC_R // H


def _sc_aggregate(gidx, sidx, fchunks, zrows):
  mesh = plsc.VectorSubcoreMesh(core_axis_name="c", subcore_axis_name="s")

  @functools.partial(
      pl.kernel,
      out_type=[
          jax.ShapeDtypeStruct((NC, NP, ACC_R, H), jnp.float32),
          jax.ShapeDtypeStruct((NC, NS, CR, H), jnp.float32),
      ],
      mesh=mesh,
      compiler_params=pltpu.CompilerParams(use_tc_tiling_on_sc=False,
                                           needs_layout_passes=False),
      scratch_types=[
          pltpu.VMEM_SHARED((ACC_R, H), jnp.float32),
          pltpu.VMEM((IGRP, B), jnp.int32),
          pltpu.VMEM((IGRP, B), jnp.int32),
          pltpu.VMEM((B, HW), jnp.float32),
          pltpu.VMEM((B, HW), jnp.float32),
          pltpu.VMEM((B // 2, H), jnp.float32),
          pltpu.VMEM((B // 2, H), jnp.float32),
          pltpu.VMEM((CR, H), jnp.float32),
          pltpu.SemaphoreType.DMA,
          pltpu.SemaphoreType.DMA,
      ],
  )
  def body(gidx_h, sidx_h, f0_h, f1_h, zrows_h,
           aggs_o, cnts_o, acc_sh, idxg_all, idxs_all,
           frg0, frg1, fh0, fh1, cnt_loc, gsem, ssem):
    c = lax.axis_index("c")
    s = lax.axis_index("s")
    rbase = s * RPT

    ones_v = jnp.full((16,), 1.0, jnp.float32)

    def upconvert(frg, half, fhb):
      @pl.loop(0, B // 2, unroll=4)
      def conv(r):
        for k in range(HW // 16):
          w = plsc.bitcast(frg[half * (B // 2) + r, pl.ds(16 * k, 16)],
                           jnp.int32)
          lo = plsc.bitcast(w << 16, jnp.float32)
          hi = plsc.bitcast(w & jnp.int32(-65536), jnp.float32)
          fhb[r, pl.ds(32 * k, 16)] = lo
          fhb[r, pl.ds(32 * k + 16, 16)] = hi

    def count(jj):
      for q in range(B // 16):
        iv = idxs_all[jj, pl.ds(16 * q, 16)]
        plsc.addupdate_scatter(
            cnt_loc, [iv >> 7, iv & jnp.int32(127)], ones_v)

    for h, fsrc in enumerate((f0_h, f1_h)):
      HB = B // 2
      pltpu.sync_copy(zrows_h.at[pl.ds(0, HB)], fh0)
      for j in range(RPT // HB):
        pltpu.sync_copy(fh0, acc_sh.at[pl.ds(rbase + j * HB, HB)])
      if h == 0:
        pltpu.sync_copy(zrows_h.at[pl.ds(0, CR)], cnt_loc)
      plsc.subcore_barrier()

      @pl.loop(0, NB // IGRP)
      def igrp_loop(ig):
        bbase = s * NB + ig * IGRP
        pltpu.sync_copy(gidx_h.at[c, pl.ds(bbase, IGRP)], idxg_all)
        pltpu.sync_copy(sidx_h.at[c, pl.ds(bbase, IGRP)], idxs_all)

        @pl.loop(0, IGRP, step=2)
        def grp(j):
          g0 = pltpu.async_copy(fsrc.at[idxg_all.at[j]], frg0, gsem)
          g1 = pltpu.async_copy(fsrc.at[idxg_all.at[j + 1]], frg1, gsem)
          prev = []
          for jj, frg, gcp in ((j, frg0, g0), (j + 1, frg1, g1)):
            gcp.wait()
            cur = []
            for half, fhb in ((0, fh0), (1, fh1)):
              if prev:
                prev.pop(0).wait()
              upconvert(frg, half, fhb)
              cur.append(pltpu.async_copy(
                  fhb,
                  acc_sh.at[idxs_all.at[jj, pl.ds(half * (B // 2), B // 2)]],
                  ssem, add=True))
            if h == 0:
              count(jj)
            prev = cur
          for cp in prev:
            cp.wait()

      plsc.subcore_barrier()
      pltpu.sync_copy(acc_sh.at[pl.ds(rbase, RPT)],
                      aggs_o.at[c, h, pl.ds(rbase, RPT)])

    pltpu.sync_copy(cnt_loc, cnts_o.at[c, s])

  return body(gidx, sidx, *fchunks, zrows)


def _tc_combine(aggs, cnts, Wt, bstack):
  RB = 256
  grid = (ACC_R // RB,)

  def body(agg_ref, cnt_ref, wt_ref, b_ref, out_ref):
    cf = cnt_ref[0].sum(axis=0).reshape(RB, 1)
    cb = cnt_ref[1].sum(axis=0).reshape(RB, 1)
    acc = cf * b_ref[0:1, :] + cb * b_ref[1:2, :]
    for ci in range(NC):
      for h in range(NP):
        acc += jnp.dot(agg_ref[ci, h],
                       wt_ref[(ci * NP + h) * H:(ci * NP + h + 1) * H],
                       preferred_element_type=jnp.float32)
    deg = cf + cb
    deg = jnp.where(deg == 0.0, 1.0, deg)
    out_ref[...] = jnp.maximum(acc / deg, 0.0)

  return pl.pallas_call(
      body,
      grid=grid,
      in_specs=[
          pl.BlockSpec((NC, NP, RB, H), lambda i: (0, 0, i, 0)),
          pl.BlockSpec((NC, NS, RB), lambda i: (0, 0, i)),
          pl.BlockSpec((2 * D, D), lambda i: (0, 0)),
          pl.BlockSpec((2, D), lambda i: (0, 0)),
      ],
      out_specs=pl.BlockSpec((RB, D), lambda i: (i, 0)),
      out_shape=jax.ShapeDtypeStruct((ACC_R, D), jnp.float32),
  )(aggs, cnts, Wt, bstack)


def _col_perm():
  p = []
  for k in range(H // 32):
    base = 32 * k
    p += [base + 2 * j for j in range(16)]
    p += [base + 2 * j + 1 for j in range(16)]
  return np.array(p)


def kernel(feat, edge_index, Wf, bf, Wb, bb):
  src = edge_index[0]
  dst = edge_index[1]
  npad = EP - E
  pad0 = jnp.zeros((npad,), jnp.int32)
  padN = jnp.full((npad,), N, jnp.int32)
  gidx = jnp.stack([jnp.concatenate([src, pad0]),
                    jnp.concatenate([dst, pad0])]).reshape(NC, EP // B, B)
  sidx = jnp.stack([jnp.concatenate([dst, padN]),
                    jnp.concatenate([src, padN])]).reshape(NC, EP // B, B)
  featb = feat.astype(jnp.bfloat16)
  fchunks = [
      lax.bitcast_convert_type(
          featb[:, i * H:(i + 1) * H].reshape(N, HW, 2), jnp.float32)
      for i in range(NP)
  ]
  zrows = jnp.zeros((B, H), jnp.float32)

  aggs, cnts = _sc_aggregate(gidx, sidx, fchunks, zrows)
  cnts = cnts.reshape(NC, NS, ACC_R)

  perm = _col_perm()
  row_order = np.concatenate(
      [ci * D + h * H + perm for ci in range(NC) for h in range(NP)])
  Wt = jnp.concatenate([Wf.T, Wb.T], axis=0)[row_order]
  bstack = jnp.stack([bf, bb])
  out = _tc_combine(aggs, cnts, Wt, bstack)
  return out[:N]


# speedup vs baseline: 1.2413x; 1.0698x over previous
import functools

import jax
import jax.numpy as jnp
import numpy as np
from jax import lax
from jax.experimental import pallas as pl
from jax.experimental.pallas import tpu as pltpu
from jax.experimental.pallas import tpu_sc as plsc

N = 10000
E = 160000
D = 256
H = 128
HW = H // 2
NP = D // H
NC = 2
NS = 16
B = 128
TPW = 10240
EP = NS * TPW
NB = TPW // B
IGRP = 8
GR = 3
ACC_R = 10240
RPT = ACC_R // NS
CR = ACC_R // H


def _sc_aggregate(gidx, sidx, fchunks, zrows):
  mesh = plsc.VectorSubcoreMesh(core_axis_name="c", subcore_axis_name="s")

  @functools.partial(
      pl.kernel,
      out_type=jax.ShapeDtypeStruct((NC, NP, ACC_R, H), jnp.float32),
      mesh=mesh,
      compiler_params=pltpu.CompilerParams(use_tc_tiling_on_sc=False,
                                           needs_layout_passes=False),
      scratch_types=[
          pltpu.VMEM_SHARED((ACC_R, H), jnp.float32),
          pltpu.VMEM((IGRP, B), jnp.int32),
          pltpu.VMEM((IGRP, B), jnp.int32),
          pltpu.VMEM((B, HW), jnp.float32),
          pltpu.VMEM((B, HW), jnp.float32),
          pltpu.VMEM((B, HW), jnp.float32),
          pltpu.VMEM((B // 2, H), jnp.float32),
          pltpu.VMEM((B // 2, H), jnp.float32),
          pltpu.SemaphoreType.DMA,
          pltpu.SemaphoreType.DMA,
      ],
  )
  def body(gidx_h, sidx_h, f0_h, f1_h, zrows_h,
           aggs_o, acc_sh, idxg_all, idxs_all,
           frg0, frg1, frg2, fh0, fh1, gsem, ssem):
    c = lax.axis_index("c")
    s = lax.axis_index("s")
    rbase = s * RPT
    frgs = (frg0, frg1, frg2)

    def upconvert(frg, half, fhb):
      @pl.loop(0, B // 2, unroll=4)
      def conv(r):
        for k in range(HW // 16):
          w = plsc.bitcast(frg[half * (B // 2) + r, pl.ds(16 * k, 16)],
                           jnp.int32)
          lo = plsc.bitcast(w << 16, jnp.float32)
          hi = plsc.bitcast(w & jnp.int32(-65536), jnp.float32)
          fhb[r, pl.ds(32 * k, 16)] = lo
          fhb[r, pl.ds(32 * k + 16, 16)] = hi

    for h, fsrc in enumerate((f0_h, f1_h)):
      HB = B // 2
      pltpu.sync_copy(zrows_h.at[pl.ds(0, HB)], fh0)
      for j in range(RPT // HB):
        pltpu.sync_copy(fh0, acc_sh.at[pl.ds(rbase + j * HB, HB)])
      plsc.subcore_barrier()

      @pl.loop(0, NB // IGRP)
      def igrp_loop(ig):
        bbase = s * NB + ig * IGRP
        pltpu.sync_copy(gidx_h.at[c, pl.ds(bbase, IGRP)], idxg_all)
        pltpu.sync_copy(sidx_h.at[c, pl.ds(bbase, IGRP)], idxs_all)

        gcps = [pltpu.async_copy(fsrc.at[idxg_all.at[j]], frgs[j], gsem)
                for j in range(GR)]
        prev = []
        for j in range(IGRP):
          gcps[j % GR].wait()
          frg = frgs[j % GR]
          cur = []
          for half, fhb in ((0, fh0), (1, fh1)):
            if prev:
              prev.pop(0).wait()
            upconvert(frg, half, fhb)
            cur.append(pltpu.async_copy(
                fhb,
                acc_sh.at[idxs_all.at[j, pl.ds(half * (B // 2), B // 2)]],
                ssem, add=True))
          prev = cur
          if j + GR < IGRP:
            gcps[(j + GR) % GR] = pltpu.async_copy(
                fsrc.at[idxg_all.at[j + GR]], frgs[(j + GR) % GR], gsem)
        for cp in prev:
          cp.wait()

      plsc.subcore_barrier()
      pltpu.sync_copy(acc_sh.at[pl.ds(rbase, RPT)],
                      aggs_o.at[c, h, pl.ds(rbase, RPT)])

  return body(gidx, sidx, *fchunks, zrows)


def _sc_counts(sidx4, zcnt):
  mesh = plsc.VectorSubcoreMesh(core_axis_name="c", subcore_axis_name="s")
  CHK = TPW // B // 10

  @functools.partial(
      pl.kernel,
      out_type=jax.ShapeDtypeStruct((NC, NS, CR, H), jnp.float32),
      mesh=mesh,
      compiler_params=pltpu.CompilerParams(use_tc_tiling_on_sc=False,
                                           needs_layout_passes=False),
      scratch_types=[
          pltpu.VMEM((CHK, B), jnp.int32),
          pltpu.VMEM((CR, H), jnp.float32),
      ],
  )
  def body(sidx_h, zc_h, cnts_o, idxb, cnt_loc):
    c = lax.axis_index("c")
    s = lax.axis_index("s")
    ones_v = jnp.full((16,), 1.0, jnp.float32)
    pltpu.sync_copy(zc_h, cnt_loc)

    @pl.loop(0, (TPW // B) // CHK)
    def chunk(k):
      pltpu.sync_copy(sidx_h.at[c, s, pl.ds(CHK * k, CHK)], idxb)
      for r in range(CHK):
        for q in range(B // 16):
          iv = idxb[r, pl.ds(16 * q, 16)]
          plsc.addupdate_scatter(
              cnt_loc, [iv >> 7, iv & jnp.int32(127)], ones_v)

    pltpu.sync_copy(cnt_loc, cnts_o.at[c, s])

  return body(sidx4, zcnt)


def _tc_combine(aggs, cnts, Wt, bstack):
  RB = 256
  grid = (ACC_R // RB,)

  def body(agg_ref, cnt_ref, wt_ref, b_ref, out_ref):
    cf = cnt_ref[0].sum(axis=0).reshape(RB, 1)
    cb = cnt_ref[1].sum(axis=0).reshape(RB, 1)
    acc = cf * b_ref[0:1, :] + cb * b_ref[1:2, :]
    for ci in range(NC):
      for h in range(NP):
        acc += jnp.dot(agg_ref[ci, h],
                       wt_ref[(ci * NP + h) * H:(ci * NP + h + 1) * H],
                       preferred_element_type=jnp.float32)
    deg = cf + cb
    deg = jnp.where(deg == 0.0, 1.0, deg)
    out_ref[...] = jnp.maximum(acc / deg, 0.0)

  return pl.pallas_call(
      body,
      grid=grid,
      in_specs=[
          pl.BlockSpec((NC, NP, RB, H), lambda i: (0, 0, i, 0)),
          pl.BlockSpec((NC, NS, RB), lambda i: (0, 0, i)),
          pl.BlockSpec((2 * D, D), lambda i: (0, 0)),
          pl.BlockSpec((2, D), lambda i: (0, 0)),
      ],
      out_specs=pl.BlockSpec((RB, D), lambda i: (i, 0)),
      out_shape=jax.ShapeDtypeStruct((ACC_R, D), jnp.float32),
  )(aggs, cnts, Wt, bstack)


def _col_perm():
  p = []
  for k in range(H // 32):
    base = 32 * k
    p += [base + 2 * j for j in range(16)]
    p += [base + 2 * j + 1 for j in range(16)]
  return np.array(p)


def kernel(feat, edge_index, Wf, bf, Wb, bb):
  src = edge_index[0]
  dst = edge_index[1]
  npad = EP - E
  pad0 = jnp.zeros((npad,), jnp.int32)
  padN = jnp.full((npad,), N, jnp.int32)
  gidx = jnp.stack([jnp.concatenate([src, pad0]),
                    jnp.concatenate([dst, pad0])]).reshape(NC, EP // B, B)
  sidx = jnp.stack([jnp.concatenate([dst, padN]),
                    jnp.concatenate([src, padN])]).reshape(NC, EP // B, B)
  featb = feat.astype(jnp.bfloat16)
  fchunks = [
      lax.bitcast_convert_type(
          featb[:, i * H:(i + 1) * H].reshape(N, HW, 2), jnp.float32)
      for i in range(NP)
  ]
  zrows = jnp.zeros((B, H), jnp.float32)

  aggs = _sc_aggregate(gidx, sidx, fchunks, zrows)
  sidx4 = sidx.reshape(NC, NS, TPW // B, B)
  zcnt = jnp.zeros((CR, H), jnp.float32)
  cnts = _sc_counts(sidx4, zcnt).reshape(NC, NS, ACC_R)

  perm = _col_perm()
  row_order = np.concatenate(
      [ci * D + h * H + perm for ci in range(NC) for h in range(NP)])
  Wt = jnp.concatenate([Wf.T, Wb.T], axis=0)[row_order]
  bstack = jnp.stack([bf, bb])
  out = _tc_combine(aggs, cnts, Wt, bstack)
  return out[:N]
